# baseline (speedup 1.0000x reference)
import functools

import jax
import jax.numpy as jnp
from jax import lax
from jax.experimental import pallas as pl
from jax.experimental.pallas import tpu as pltpu
from jax.experimental.pallas import tpu_sc as plsc

VOCAB = 100000
EMB = 300
B = 1024
L = 150
F = 150
FP = 256
NUM_CLASSES = 1000
EPS = 1e-5
TOK = B * L
SCALE = 16384.0

NC = 2
NS = 16
NW = NC * NS
NSLICE = 4
BS = B // NSLICE
TOKS = BS * L
PER_W = TOKS // NW
CH = 120
NCH = PER_W // CH
NBUF = 4


VB = 10000


def _proj_body(t_ref, w_ref, b_ref, o_ref):
    acc = lax.dot_general(
        t_ref[...], w_ref[...], (((1,), (1,)), ((), ())),
        preferred_element_type=jnp.float32) + b_ref[...]
    o_ref[...] = acc


def _project(table, w1p, b1p):
    return pl.pallas_call(
        _proj_body,
        grid=(VOCAB // VB,),
        in_specs=[
            pl.BlockSpec((VB, EMB), lambda i: (i, 0)),
            pl.BlockSpec((FP, EMB), lambda i: (0, 0)),
            pl.BlockSpec((1, FP), lambda i: (0, 0)),
        ],
        out_specs=pl.BlockSpec((VB, FP), lambda i: (i, 0)),
        out_shape=jax.ShapeDtypeStruct((VOCAB, FP), jnp.float32),
    )(table, w1p, b1p)


def _sc_gather_build():
    mesh = plsc.VectorSubcoreMesh(core_axis_name="c", subcore_axis_name="s")

    @functools.partial(
        pl.kernel,
        mesh=mesh,
        out_type=jax.ShapeDtypeStruct((TOKS, FP), jnp.float32),
        scratch_types=(
            [pltpu.VMEM((NCH, CH), jnp.int32)]
            + [pltpu.VMEM((CH, FP), jnp.float32) for _ in range(NBUF)]
            + [pltpu.SemaphoreType.DMA for _ in range(2 * NBUF)]
        ),
    )
    def gather(idx_hbm, p_hbm, out_hbm, idx_v, *rest):
        bufs = rest[:NBUF]
        gsems = rest[NBUF:2 * NBUF]
        osems = rest[2 * NBUF:]
        wid = lax.axis_index("s") * NC + lax.axis_index("c")
        base = wid * PER_W
        pltpu.sync_copy(idx_hbm.at[wid], idx_v)

        g_h = {}
        o_h = {}
        for c in range(NCH):
            b = c % NBUF
            if c >= NBUF:
                o_h[c - NBUF].wait()
            g_h[c] = pltpu.async_copy(p_hbm.at[idx_v.at[c]], bufs[b],
                                      gsems[b])
            j = c - (NBUF - 1)
            if j >= 0:
                g_h[j].wait()
                o_h[j] = pltpu.async_copy(
                    bufs[j % NBUF], out_hbm.at[pl.ds(base + j * CH, CH)],
                    osems[j % NBUF])
        for j in range(NCH - NBUF + 1, NCH):
            g_h[j].wait()
            o_h[j] = pltpu.async_copy(
                bufs[j % NBUF], out_hbm.at[pl.ds(base + j * CH, CH)],
                osems[j % NBUF])
        for j in range(NCH - NBUF, NCH):
            o_h[j].wait()

    return gather


_sc_gather_cache = []


def _sc_gather():
    if not _sc_gather_cache:
        _sc_gather_cache.append(_sc_gather_build())
    return _sc_gather_cache[0]


BB = 64


def _stats_body(a0_ref, q0_ref, g_ref, a_ref, q_ref):
    @pl.when(pl.program_id(0) == 0)
    def _init():
        a_ref[...] = a0_ref[...]
        q_ref[...] = q0_ref[...]

    g = g_ref[...]
    a_ref[...] += jnp.sum(g, axis=0)
    q_ref[...] += jnp.sum(g * g, axis=0)


def _stats(a0, q0, g3):
    return pl.pallas_call(
        _stats_body,
        grid=(BS // BB,),
        in_specs=[
            pl.BlockSpec((L, FP), lambda i: (0, 0)),
            pl.BlockSpec((L, FP), lambda i: (0, 0)),
            pl.BlockSpec((BB, L, FP), lambda i: (i, 0, 0)),
        ],
        out_specs=[
            pl.BlockSpec((L, FP), lambda i: (0, 0)),
            pl.BlockSpec((L, FP), lambda i: (0, 0)),
        ],
        out_shape=[
            jax.ShapeDtypeStruct((L, FP), jnp.float32),
            jax.ShapeDtypeStruct((L, FP), jnp.float32),
        ],
    )(a0, q0, g3)


def _norm_body(a_ref, q_ref, g1_ref, be1_ref, g_ref, o_ref):
    denom = float(B * F)
    s1 = jnp.sum(a_ref[...], axis=1, keepdims=True)
    s2 = jnp.sum(q_ref[...], axis=1, keepdims=True)
    m = s1 / denom
    var = s2 / denom - m * m
    scale = lax.rsqrt(var + EPS) * g1_ref[...]
    shift = be1_ref[...] - m * scale
    g = g_ref[...]
    t = jnp.maximum(g * scale[None] + shift[None], 0.0)
    o_ref[...] = jnp.sum(t, axis=1) * (1.0 / L)


def _normpool(a, q, g1c, be1c, g3):
    return pl.pallas_call(
        _norm_body,
        grid=(BS // BB,),
        in_specs=[
            pl.BlockSpec((L, FP), lambda i: (0, 0)),
            pl.BlockSpec((L, FP), lambda i: (0, 0)),
            pl.BlockSpec((L, 1), lambda i: (0, 0)),
            pl.BlockSpec((L, 1), lambda i: (0, 0)),
            pl.BlockSpec((BB, L, FP), lambda i: (i, 0, 0)),
        ],
        out_specs=pl.BlockSpec((BB, FP), lambda i: (i, 0)),
        out_shape=jax.ShapeDtypeStruct((BS, FP), jnp.float32),
    )(a, q, g1c, be1c, g3)


def _tail_body(avg_ref, w2_ref, b2_ref, g2_ref, be2_ref, w3_ref, b3_ref,
               o_ref):
    z = lax.dot_general(avg_ref[...], w2_ref[...], (((1,), (1,)), ((), ())),
                        preferred_element_type=jnp.float32) + b2_ref[...]
    m = jnp.mean(z, axis=0, keepdims=True)
    var = jnp.mean(z * z, axis=0, keepdims=True) - m * m
    zn = (z - m) * lax.rsqrt(var + EPS) * g2_ref[...] + be2_ref[...]
    zn = jnp.maximum(zn, 0.0)
    o_ref[...] = lax.dot_general(zn, w3_ref[...], (((1,), (1,)), ((), ())),
                                 preferred_element_type=jnp.float32
                                 ) + b3_ref[...]


def _tail(avg, w2p, b2r, g2r, be2r, w3, b3r):
    return pl.pallas_call(
        _tail_body,
        out_shape=jax.ShapeDtypeStruct((B, NUM_CLASSES), jnp.float32),
    )(avg, w2p, b2r, g2r, be2r, w3, b3r)


def kernel(x, table, W1, b1, g1, be1, W2, b2, g2, be2, W3, b3):
    w1p = jnp.pad(W1, ((0, FP - F), (0, 0)))
    b1p = jnp.pad(b1, (0, FP - F)).reshape(1, FP)
    p = _project(table, w1p, b1p)

    idx_all = x.astype(jnp.int32).reshape(NSLICE, NW, NCH, CH)
    zero = jnp.zeros((L, FP), jnp.float32)
    gath = _sc_gather()
    g3s = []
    a, q = zero, zero
    for s in range(NSLICE):
        g3 = gath(idx_all[s], p).reshape(BS, L, FP)
        g3s.append(g3)
        a, q = _stats(a, q, g3)

    g1c = g1.reshape(L, 1)
    be1c = be1.reshape(L, 1)
    avg = jnp.concatenate(
        [_normpool(a, q, g1c, be1c, g3) for g3 in g3s], axis=0)

    w2p = jnp.pad(W2, ((0, 0), (0, FP - F)))
    out = _tail(avg, w2p, b2.reshape(1, -1), g2.reshape(1, -1),
                be2.reshape(1, -1), W3, b3.reshape(1, -1))
    return out


# speedup vs baseline: 1.0134x; 1.0134x over previous
import functools

import jax
import jax.numpy as jnp
from jax import lax
from jax.experimental import pallas as pl
from jax.experimental.pallas import tpu as pltpu
from jax.experimental.pallas import tpu_sc as plsc

VOCAB = 100000
EMB = 300
B = 1024
L = 150
F = 150
FP = 256
NUM_CLASSES = 1000
EPS = 1e-5
TOK = B * L
SCALE = 16384.0

NC = 2
NS = 16
NW = NC * NS
NSLICE = 2
BS = B // NSLICE
TOKS = BS * L
PER_W = TOKS // NW
CH = 120
NCH = PER_W // CH
NBUF = 4


VB = 10000


def _proj_body(t_ref, w_ref, b_ref, o_ref):
    acc = lax.dot_general(
        t_ref[...], w_ref[...], (((1,), (1,)), ((), ())),
        preferred_element_type=jnp.float32) + b_ref[...]
    o_ref[...] = acc


def _project(table, w1p, b1p):
    return pl.pallas_call(
        _proj_body,
        grid=(VOCAB // VB,),
        in_specs=[
            pl.BlockSpec((VB, EMB), lambda i: (i, 0)),
            pl.BlockSpec((FP, EMB), lambda i: (0, 0)),
            pl.BlockSpec((1, FP), lambda i: (0, 0)),
        ],
        out_specs=pl.BlockSpec((VB, FP), lambda i: (i, 0)),
        out_shape=jax.ShapeDtypeStruct((VOCAB, FP), jnp.float32),
    )(table, w1p, b1p)


def _sc_gather_build():
    mesh = plsc.VectorSubcoreMesh(core_axis_name="c", subcore_axis_name="s")

    @functools.partial(
        pl.kernel,
        mesh=mesh,
        out_type=jax.ShapeDtypeStruct((TOKS, FP), jnp.float32),
        scratch_types=(
            [pltpu.VMEM((NCH, CH), jnp.int32)]
            + [pltpu.VMEM((CH, FP), jnp.float32) for _ in range(NBUF)]
            + [pltpu.SemaphoreType.DMA for _ in range(2 * NBUF)]
        ),
    )
    def gather(idx_hbm, p_hbm, out_hbm, idx_v, *rest):
        bufs = rest[:NBUF]
        gsems = rest[NBUF:2 * NBUF]
        osems = rest[2 * NBUF:]
        wid = lax.axis_index("s") * NC + lax.axis_index("c")
        base = wid * PER_W
        pltpu.sync_copy(idx_hbm.at[wid], idx_v)

        g_h = {}
        o_h = {}
        for c in range(NCH):
            b = c % NBUF
            if c >= NBUF:
                o_h[c - NBUF].wait()
            g_h[c] = pltpu.async_copy(p_hbm.at[idx_v.at[c]], bufs[b],
                                      gsems[b])
            j = c - (NBUF - 1)
            if j >= 0:
                g_h[j].wait()
                o_h[j] = pltpu.async_copy(
                    bufs[j % NBUF], out_hbm.at[pl.ds(base + j * CH, CH)],
                    osems[j % NBUF])
        for j in range(NCH - NBUF + 1, NCH):
            g_h[j].wait()
            o_h[j] = pltpu.async_copy(
                bufs[j % NBUF], out_hbm.at[pl.ds(base + j * CH, CH)],
                osems[j % NBUF])
        for j in range(NCH - NBUF, NCH):
            o_h[j].wait()

    return gather


_sc_gather_cache = []


def _sc_gather():
    if not _sc_gather_cache:
        _sc_gather_cache.append(_sc_gather_build())
    return _sc_gather_cache[0]


BB = 64


def _stats_body(a0_ref, q0_ref, g_ref, a_ref, q_ref):
    @pl.when(pl.program_id(0) == 0)
    def _init():
        a_ref[...] = a0_ref[...]
        q_ref[...] = q0_ref[...]

    g = g_ref[...]
    a_ref[...] += jnp.sum(g, axis=0)
    q_ref[...] += jnp.sum(g * g, axis=0)


def _stats(a0, q0, g3):
    return pl.pallas_call(
        _stats_body,
        grid=(BS // BB,),
        in_specs=[
            pl.BlockSpec((L, FP), lambda i: (0, 0)),
            pl.BlockSpec((L, FP), lambda i: (0, 0)),
            pl.BlockSpec((BB, L, FP), lambda i: (i, 0, 0)),
        ],
        out_specs=[
            pl.BlockSpec((L, FP), lambda i: (0, 0)),
            pl.BlockSpec((L, FP), lambda i: (0, 0)),
        ],
        out_shape=[
            jax.ShapeDtypeStruct((L, FP), jnp.float32),
            jax.ShapeDtypeStruct((L, FP), jnp.float32),
        ],
    )(a0, q0, g3)


def _norm_body(a_ref, q_ref, g1_ref, be1_ref, g_ref, o_ref):
    denom = float(B * F)
    s1 = jnp.sum(a_ref[...], axis=1, keepdims=True)
    s2 = jnp.sum(q_ref[...], axis=1, keepdims=True)
    m = s1 / denom
    var = s2 / denom - m * m
    scale = lax.rsqrt(var + EPS) * g1_ref[...]
    shift = be1_ref[...] - m * scale
    g = g_ref[...]
    t = jnp.maximum(g * scale[None] + shift[None], 0.0)
    o_ref[...] = jnp.sum(t, axis=1) * (1.0 / L)


def _normpool(a, q, g1c, be1c, g3):
    return pl.pallas_call(
        _norm_body,
        grid=(BS // BB,),
        in_specs=[
            pl.BlockSpec((L, FP), lambda i: (0, 0)),
            pl.BlockSpec((L, FP), lambda i: (0, 0)),
            pl.BlockSpec((L, 1), lambda i: (0, 0)),
            pl.BlockSpec((L, 1), lambda i: (0, 0)),
            pl.BlockSpec((BB, L, FP), lambda i: (i, 0, 0)),
        ],
        out_specs=pl.BlockSpec((BB, FP), lambda i: (i, 0)),
        out_shape=jax.ShapeDtypeStruct((BS, FP), jnp.float32),
    )(a, q, g1c, be1c, g3)


def _tail_body(avg_ref, w2_ref, b2_ref, g2_ref, be2_ref, w3_ref, b3_ref,
               o_ref):
    z = lax.dot_general(avg_ref[...], w2_ref[...], (((1,), (1,)), ((), ())),
                        preferred_element_type=jnp.float32) + b2_ref[...]
    m = jnp.mean(z, axis=0, keepdims=True)
    var = jnp.mean(z * z, axis=0, keepdims=True) - m * m
    zn = (z - m) * lax.rsqrt(var + EPS) * g2_ref[...] + be2_ref[...]
    zn = jnp.maximum(zn, 0.0)
    o_ref[...] = lax.dot_general(zn, w3_ref[...], (((1,), (1,)), ((), ())),
                                 preferred_element_type=jnp.float32
                                 ) + b3_ref[...]


def _tail(avg, w2p, b2r, g2r, be2r, w3, b3r):
    return pl.pallas_call(
        _tail_body,
        out_shape=jax.ShapeDtypeStruct((B, NUM_CLASSES), jnp.float32),
    )(avg, w2p, b2r, g2r, be2r, w3, b3r)


def kernel(x, table, W1, b1, g1, be1, W2, b2, g2, be2, W3, b3):
    w1p = jnp.pad(W1, ((0, FP - F), (0, 0)))
    b1p = jnp.pad(b1, (0, FP - F)).reshape(1, FP)
    p = _project(table, w1p, b1p)

    idx_all = x.astype(jnp.int32).reshape(NSLICE, NW, NCH, CH)
    zero = jnp.zeros((L, FP), jnp.float32)
    gath = _sc_gather()
    g3s = []
    a, q = zero, zero
    for s in range(NSLICE):
        g3 = gath(idx_all[s], p).reshape(BS, L, FP)
        g3s.append(g3)
        a, q = _stats(a, q, g3)

    g1c = g1.reshape(L, 1)
    be1c = be1.reshape(L, 1)
    avg = jnp.concatenate(
        [_normpool(a, q, g1c, be1c, g3) for g3 in g3s], axis=0)

    w2p = jnp.pad(W2, ((0, 0), (0, FP - F)))
    out = _tail(avg, w2p, b2.reshape(1, -1), g2.reshape(1, -1),
                be2.reshape(1, -1), W3, b3.reshape(1, -1))
    return out
